# baseline (speedup 1.0000x reference)
import functools

import jax
import jax.numpy as jnp
from jax.experimental import pallas as pl
from jax.experimental.pallas import tpu as pltpu

_UNROLL = 4

_DN_T = (((1,), (1,)), ((), ()))


def _fused_kernel(x_ref, wgih_ref, bg_ref, bc_ref, lf_ref, rf_ref,
                  wglhh_ref, wgrhh_ref, wcih_ref, wclhh_ref, wcrhh_ref,
                  out_ref, xg_scr, xc_scr, wg_scr, wc_scr, h_scr, lrh_scr,
                  lrhb_scr, *, L, B, H):
    wg_scr[0:H, :] = wglhh_ref[:, :].T.astype(jnp.bfloat16)
    wg_scr[H:2 * H, :] = wgrhh_ref[:, :].T.astype(jnp.bfloat16)
    wc_scr[0:H, :] = wclhh_ref[:, :].T.astype(jnp.bfloat16)
    wc_scr[H:2 * H, :] = wcrhh_ref[:, :].T.astype(jnp.bfloat16)
    x = x_ref[:, :]
    xg_scr[:, :] = jax.lax.dot_general(
        x, wgih_ref[:, :], dimension_numbers=_DN_T,
        preferred_element_type=jnp.float32) + bg_ref[:, :]
    xc_scr[:, :] = jax.lax.dot_general(
        x, wcih_ref[:, :], dimension_numbers=_DN_T,
        preferred_element_type=jnp.float32) + bc_ref[:, :]
    h_scr[pl.ds(L * B, B), :] = jnp.zeros((B, H), jnp.float32)

    def step(t, u):
        o = u * B
        for b in range(B):
            li = lf_ref[t, b]
            ri = rf_ref[t, b]
            lrh_scr[pl.ds(o + b, 1), 0:H] = h_scr[pl.ds(li, 1), :]
            lrh_scr[pl.ds(o + b, 1), H:2 * H] = h_scr[pl.ds(ri, 1), :]
        lrh = lrh_scr[o:o + B, :]
        lh = lrh[:, 0:H]
        rh = lrh[:, H:2 * H]
        lrhb_scr[o:o + B, :] = lrh.astype(jnp.bfloat16)
        gates = jax.nn.sigmoid(
            xg_scr[pl.ds(t * B, B), :]
            + jnp.dot(lrhb_scr[o:o + B, :], wg_scr[:, :],
                      preferred_element_type=jnp.float32)
        )
        rl = gates[:, 0:H]
        rr = gates[:, H:2 * H]
        zl = gates[:, 2 * H:3 * H]
        zr = gates[:, 3 * H:4 * H]
        z = gates[:, 4 * H:5 * H]
        lrhb_scr[o:o + B, 0:H] = (rl * lh).astype(jnp.bfloat16)
        lrhb_scr[o:o + B, H:2 * H] = (rr * rh).astype(jnp.bfloat16)
        cell = jnp.tanh(
            xc_scr[pl.ds(t * B, B), :]
            + jnp.dot(lrhb_scr[o:o + B, :], wc_scr[:, :],
                      preferred_element_type=jnp.float32)
        )
        h = zl * lh + zr * rh + z * cell
        h_scr[pl.ds(t * B, B), :] = h
        out_ref[:, pl.ds(t, 1), :] = h.reshape(B, 1, H)

    def body(i, carry):
        for u in range(_UNROLL):
            step(i * _UNROLL + u, u)
        return carry

    jax.lax.fori_loop(0, L // _UNROLL, body, 0)


def kernel(inputs, left_idx, right_idx, W_gih, b_gih, W_glhh, W_grhh,
           W_cih, b_cih, W_clhh, W_crhh):
    L, B, D = inputs.shape
    H = W_cih.shape[0]

    x_flat = inputs.reshape(L * B, D)
    tvec = jnp.arange(L, dtype=jnp.int32)[:, None]
    bvec = jnp.arange(B, dtype=jnp.int32)[None, :]
    lf = jnp.where(left_idx < tvec,
                   jnp.clip(left_idx, 0, L - 1) * B + bvec, L * B + bvec)
    rf = jnp.where(right_idx < tvec,
                   jnp.clip(right_idx, 0, L - 1) * B + bvec, L * B + bvec)

    vm = pl.BlockSpec(memory_space=pltpu.VMEM)
    sm = pl.BlockSpec(memory_space=pltpu.SMEM)

    out = pl.pallas_call(
        functools.partial(_fused_kernel, L=L, B=B, H=H),
        in_specs=[vm, vm, vm, vm, sm, sm, vm, vm, vm, vm, vm],
        out_specs=vm,
        out_shape=jax.ShapeDtypeStruct((B, L, H), jnp.float32),
        scratch_shapes=[
            pltpu.VMEM((L * B, 5 * H), jnp.float32),
            pltpu.VMEM((L * B, H), jnp.float32),
            pltpu.VMEM((2 * H, 5 * H), jnp.bfloat16),
            pltpu.VMEM((2 * H, H), jnp.bfloat16),
            pltpu.VMEM((L * B + B, H), jnp.float32),
            pltpu.VMEM((_UNROLL * B, 2 * H), jnp.float32),
            pltpu.VMEM((_UNROLL * B, 2 * H), jnp.bfloat16),
        ],
    )(x_flat, W_gih, b_gih[None, :], b_cih[None, :], lf, rf,
      W_glhh, W_grhh, W_cih, W_clhh, W_crhh)

    return out


# speedup vs baseline: 1.0113x; 1.0113x over previous
import functools

import jax
import jax.numpy as jnp
from jax.experimental import pallas as pl
from jax.experimental.pallas import tpu as pltpu

_UNROLL = 8

_DN_T = (((1,), (1,)), ((), ()))


def _fused_kernel(x_ref, wgih_ref, bg_ref, bc_ref, lf_ref, rf_ref,
                  wglhh_ref, wgrhh_ref, wcih_ref, wclhh_ref, wcrhh_ref,
                  out_ref, xg_scr, xc_scr, wg_scr, wc_scr, wx_scr, h_scr,
                  lrh_scr, *, L, B, H, D):
    wg_scr[0:H, :] = wglhh_ref[:, :].T
    wg_scr[H:2 * H, :] = wgrhh_ref[:, :].T
    wc_scr[0:H, :] = wclhh_ref[:, :].T
    wc_scr[H:2 * H, :] = wcrhh_ref[:, :].T
    wx_scr[:, :] = wgih_ref[:, :].T
    x = x_ref[:, :]
    xg_scr[:, :] = jnp.dot(
        x, wx_scr[:, :], preferred_element_type=jnp.float32) + bg_ref[:, :]
    xc_scr[:, :] = jax.lax.dot_general(
        x, wcih_ref[:, :], dimension_numbers=_DN_T,
        preferred_element_type=jnp.float32) + bc_ref[:, :]
    h_scr[pl.ds(L * B, B), :] = jnp.zeros((B, H), jnp.float32)

    def step(t, u):
        o = u * B
        for b in range(B):
            li = lf_ref[t, b]
            ri = rf_ref[t, b]
            lrh_scr[pl.ds(o + b, 1), 0:H] = h_scr[pl.ds(li, 1), :]
            lrh_scr[pl.ds(o + b, 1), H:2 * H] = h_scr[pl.ds(ri, 1), :]
        lrh = lrh_scr[o:o + B, :]
        lh = lrh[:, 0:H]
        rh = lrh[:, H:2 * H]
        gates = jax.nn.sigmoid(
            xg_scr[pl.ds(t * B, B), :]
            + jnp.dot(lrh, wg_scr[:, :], preferred_element_type=jnp.float32)
        )
        rl = gates[:, 0:H]
        rr = gates[:, H:2 * H]
        zl = gates[:, 2 * H:3 * H]
        zr = gates[:, 3 * H:4 * H]
        z = gates[:, 4 * H:5 * H]
        lrh_scr[o:o + B, 0:H] = rl * lh
        lrh_scr[o:o + B, H:2 * H] = rr * rh
        cell = jnp.tanh(
            xc_scr[pl.ds(t * B, B), :]
            + jnp.dot(lrh_scr[o:o + B, :], wc_scr[:, :],
                      preferred_element_type=jnp.float32)
        )
        h = zl * lh + zr * rh + z * cell
        h_scr[pl.ds(t * B, B), :] = h
        out_ref[:, pl.ds(t, 1), :] = h.reshape(B, 1, H)

    def body(i, carry):
        for u in range(_UNROLL):
            step(i * _UNROLL + u, u)
        return carry

    jax.lax.fori_loop(0, L // _UNROLL, body, 0)


def kernel(inputs, left_idx, right_idx, W_gih, b_gih, W_glhh, W_grhh,
           W_cih, b_cih, W_clhh, W_crhh):
    L, B, D = inputs.shape
    H = W_cih.shape[0]

    x_flat = inputs.reshape(L * B, D)
    tvec = jnp.arange(L, dtype=jnp.int32)[:, None]
    bvec = jnp.arange(B, dtype=jnp.int32)[None, :]
    lf = jnp.where(left_idx < tvec,
                   jnp.clip(left_idx, 0, L - 1) * B + bvec, L * B + bvec)
    rf = jnp.where(right_idx < tvec,
                   jnp.clip(right_idx, 0, L - 1) * B + bvec, L * B + bvec)

    vm = pl.BlockSpec(memory_space=pltpu.VMEM)
    sm = pl.BlockSpec(memory_space=pltpu.SMEM)

    out = pl.pallas_call(
        functools.partial(_fused_kernel, L=L, B=B, H=H, D=D),
        in_specs=[vm, vm, vm, vm, sm, sm, vm, vm, vm, vm, vm],
        out_specs=vm,
        out_shape=jax.ShapeDtypeStruct((B, L, H), jnp.float32),
        scratch_shapes=[
            pltpu.VMEM((L * B, 5 * H), jnp.float32),
            pltpu.VMEM((L * B, H), jnp.float32),
            pltpu.VMEM((2 * H, 5 * H), jnp.float32),
            pltpu.VMEM((2 * H, H), jnp.float32),
            pltpu.VMEM((D, 5 * H), jnp.float32),
            pltpu.VMEM((L * B + B, H), jnp.float32),
            pltpu.VMEM((_UNROLL * B, 2 * H), jnp.float32),
        ],
    )(x_flat, W_gih, b_gih[None, :], b_cih[None, :], lf, rf,
      W_glhh, W_grhh, W_cih, W_clhh, W_crhh)

    return out


# speedup vs baseline: 1.0119x; 1.0006x over previous
import functools

import jax
import jax.numpy as jnp
from jax.experimental import pallas as pl
from jax.experimental.pallas import tpu as pltpu

_UNROLL = 16

_DN_T = (((1,), (1,)), ((), ()))


def _fused_kernel(x_ref, wgih_ref, bg_ref, bc_ref, lf_ref, rf_ref,
                  wglhh_ref, wgrhh_ref, wcih_ref, wclhh_ref, wcrhh_ref,
                  out_ref, xg_scr, xc_scr, wg_scr, wc_scr, wx_scr, h_scr,
                  lrh_scr, *, L, B, H, D):
    wg_scr[0:H, :] = wglhh_ref[:, :].T
    wg_scr[H:2 * H, :] = wgrhh_ref[:, :].T
    wc_scr[0:H, :] = wclhh_ref[:, :].T
    wc_scr[H:2 * H, :] = wcrhh_ref[:, :].T
    wx_scr[:, :] = wgih_ref[:, :].T
    x = x_ref[:, :]
    xg_scr[:, :] = jnp.dot(
        x, wx_scr[:, :], preferred_element_type=jnp.float32) + bg_ref[:, :]
    xc_scr[:, :] = jax.lax.dot_general(
        x, wcih_ref[:, :], dimension_numbers=_DN_T,
        preferred_element_type=jnp.float32) + bc_ref[:, :]
    h_scr[pl.ds(L * B, B), :] = jnp.zeros((B, H), jnp.float32)

    def step(t, u):
        o = u * B
        for b in range(B):
            li = lf_ref[t, b]
            ri = rf_ref[t, b]
            lrh_scr[pl.ds(o + b, 1), 0:H] = h_scr[pl.ds(li, 1), :]
            lrh_scr[pl.ds(o + b, 1), H:2 * H] = h_scr[pl.ds(ri, 1), :]
        lrh = lrh_scr[o:o + B, :]
        lh = lrh[:, 0:H]
        rh = lrh[:, H:2 * H]
        gates = jax.nn.sigmoid(
            xg_scr[pl.ds(t * B, B), :]
            + jnp.dot(lrh, wg_scr[:, :], preferred_element_type=jnp.float32)
        )
        rl = gates[:, 0:H]
        rr = gates[:, H:2 * H]
        zl = gates[:, 2 * H:3 * H]
        zr = gates[:, 3 * H:4 * H]
        z = gates[:, 4 * H:5 * H]
        lrh_scr[o:o + B, 0:H] = rl * lh
        lrh_scr[o:o + B, H:2 * H] = rr * rh
        cell = jnp.tanh(
            xc_scr[pl.ds(t * B, B), :]
            + jnp.dot(lrh_scr[o:o + B, :], wc_scr[:, :],
                      preferred_element_type=jnp.float32)
        )
        h = zl * lh + zr * rh + z * cell
        h_scr[pl.ds(t * B, B), :] = h
        out_ref[:, pl.ds(t, 1), :] = h.reshape(B, 1, H)

    def body(i, carry):
        for u in range(_UNROLL):
            step(i * _UNROLL + u, u)
        return carry

    jax.lax.fori_loop(0, L // _UNROLL, body, 0)


def kernel(inputs, left_idx, right_idx, W_gih, b_gih, W_glhh, W_grhh,
           W_cih, b_cih, W_clhh, W_crhh):
    L, B, D = inputs.shape
    H = W_cih.shape[0]

    x_flat = inputs.reshape(L * B, D)
    tvec = jnp.arange(L, dtype=jnp.int32)[:, None]
    bvec = jnp.arange(B, dtype=jnp.int32)[None, :]
    lf = jnp.where(left_idx < tvec,
                   jnp.clip(left_idx, 0, L - 1) * B + bvec, L * B + bvec)
    rf = jnp.where(right_idx < tvec,
                   jnp.clip(right_idx, 0, L - 1) * B + bvec, L * B + bvec)

    vm = pl.BlockSpec(memory_space=pltpu.VMEM)
    sm = pl.BlockSpec(memory_space=pltpu.SMEM)

    out = pl.pallas_call(
        functools.partial(_fused_kernel, L=L, B=B, H=H, D=D),
        in_specs=[vm, vm, vm, vm, sm, sm, vm, vm, vm, vm, vm],
        out_specs=vm,
        out_shape=jax.ShapeDtypeStruct((B, L, H), jnp.float32),
        scratch_shapes=[
            pltpu.VMEM((L * B, 5 * H), jnp.float32),
            pltpu.VMEM((L * B, H), jnp.float32),
            pltpu.VMEM((2 * H, 5 * H), jnp.float32),
            pltpu.VMEM((2 * H, H), jnp.float32),
            pltpu.VMEM((D, 5 * H), jnp.float32),
            pltpu.VMEM((L * B + B, H), jnp.float32),
            pltpu.VMEM((_UNROLL * B, 2 * H), jnp.float32),
        ],
    )(x_flat, W_gih, b_gih[None, :], b_cih[None, :], lf, rf,
      W_glhh, W_grhh, W_cih, W_clhh, W_crhh)

    return out
